# baseline (speedup 1.0000x reference)
import functools
import numpy as np
import jax
import jax.numpy as jnp
from jax.experimental import pallas as pl
from jax.experimental.pallas import tpu as pltpu

N_NODES = 10000
N_PAD = 10240
EMB = 128
NUM_LAYER = 4
NUM_TASK = 10
NUM_GRAPHS = 128
N_EDGES = 320000

_DEG_HIST = np.concatenate([np.zeros(32, np.float32), np.array([10000.0], np.float32)])
_B = np.arange(_DEG_HIST.shape[0], dtype=np.float32)
AVG_LOG = float((np.log(_B + 1.0) * _DEG_HIST).sum() / _DEG_HIST.sum())

NB = 1024
N_BLK = N_PAD // NB


def _emb_body(x_ref, w_ref, b_ref, out_ref):
    out_ref[...] = jax.lax.dot_general(
        w_ref[...], x_ref[...], (((0,), (1,)), ((), ())),
        preferred_element_type=jnp.float32) + b_ref[...]


def _emb_call(x_pad, W_emb, b_col):
    return pl.pallas_call(
        _emb_body,
        grid=(N_BLK,),
        in_specs=[
            pl.BlockSpec((NB, 3), lambda i: (i, 0)),
            pl.BlockSpec((3, EMB), lambda i: (0, 0)),
            pl.BlockSpec((EMB, 1), lambda i: (0, 0)),
        ],
        out_specs=pl.BlockSpec((EMB, NB), lambda i: (0, i)),
        out_shape=jax.ShapeDtypeStruct((EMB, N_PAD), jnp.float32),
    )(x_pad, W_emb, b_col)


def _combine_body(deg_ref, agg_ref, w_ref, b_ref, h_ref, out_ref):
    deg = deg_ref[...]
    degc = jnp.maximum(deg, 1.0)
    s = agg_ref[0:EMB, :]
    mn = agg_ref[EMB:2 * EMB, :]
    mx = agg_ref[2 * EMB:3 * EMB, :]
    sq = agg_ref[3 * EMB:4 * EMB, :]
    mean = s / degc
    msq = sq / degc
    std = jnp.sqrt(jnp.maximum(msq - mean * mean, 0.0) + 1e-5)
    has = deg > 0.0
    mn = jnp.where(has, mn, 0.0)
    mx = jnp.where(has, mx, 0.0)
    aggfix = jnp.concatenate([mean, mn, mx, std], axis=0)
    A = jax.lax.dot_general(
        w_ref[...], aggfix, (((1,), (0,)), ((), ())),
        preferred_element_type=jnp.float32)
    logd = jnp.log(deg + 1.0)
    s1 = logd / AVG_LOG
    s2 = jnp.where(logd > 0.0, AVG_LOG / jnp.maximum(logd, 1e-20), 0.0)
    c = A[0:EMB, :] + s1 * A[EMB:2 * EMB, :] + s2 * A[2 * EMB:3 * EMB, :] + b_ref[...]
    out_ref[...] = jnp.maximum(c, 0.0) + h_ref[...]


def _combine_call(deg_row, aggT, wT_stack, b_col, hT):
    return pl.pallas_call(
        _combine_body,
        grid=(N_BLK,),
        in_specs=[
            pl.BlockSpec((1, NB), lambda i: (0, i)),
            pl.BlockSpec((4 * EMB, NB), lambda i: (0, i)),
            pl.BlockSpec((3 * EMB, 4 * EMB), lambda i: (0, 0)),
            pl.BlockSpec((EMB, 1), lambda i: (0, 0)),
            pl.BlockSpec((EMB, NB), lambda i: (0, i)),
        ],
        out_specs=pl.BlockSpec((EMB, NB), lambda i: (0, i)),
        out_shape=jax.ShapeDtypeStruct((EMB, N_PAD), jnp.float32),
    )(deg_row, aggT, wT_stack, b_col, hT)


def _pool_body(batch_ref, h_ref, w1_ref, b1_ref, w2_ref, b2_ref, w3_ref,
               b3_ref, out_ref, acc):
    i = pl.program_id(0)

    @pl.when(i == 0)
    def _():
        acc[...] = jnp.zeros_like(acc)

    b = batch_ref[...]
    gids = jax.lax.broadcasted_iota(jnp.int32, (NUM_GRAPHS, NB), 0)
    M = (b == gids).astype(jnp.float32)
    h_ext = jnp.concatenate(
        [h_ref[...], jnp.ones((1, NB), jnp.float32)], axis=0)
    acc[...] += jax.lax.dot_general(
        h_ext, M, (((1,), (1,)), ((), ())),
        preferred_element_type=jnp.float32)

    @pl.when(i == pl.num_programs(0) - 1)
    def _():
        a = acc[...]
        hgm = a[0:EMB, :] / jnp.maximum(a[EMB:EMB + 1, :], 1.0)
        z1 = jnp.maximum(jax.lax.dot_general(
            w1_ref[...], hgm, (((0,), (0,)), ((), ())),
            preferred_element_type=jnp.float32) + b1_ref[...], 0.0)
        z2 = jnp.maximum(jax.lax.dot_general(
            w2_ref[...], z1, (((0,), (0,)), ((), ())),
            preferred_element_type=jnp.float32) + b2_ref[...], 0.0)
        out = jax.lax.dot_general(
            z2, w3_ref[...], (((0,), (0,)), ((), ())),
            preferred_element_type=jnp.float32) + b3_ref[...]
        out_ref[...] = out


def _pool_call(batch_row, hT, W1, b1c, W2, b2c, W3, b3r):
    return pl.pallas_call(
        _pool_body,
        grid=(N_BLK,),
        in_specs=[
            pl.BlockSpec((1, NB), lambda i: (0, i)),
            pl.BlockSpec((EMB, NB), lambda i: (0, i)),
            pl.BlockSpec((EMB, EMB // 2), lambda i: (0, 0)),
            pl.BlockSpec((EMB // 2, 1), lambda i: (0, 0)),
            pl.BlockSpec((EMB // 2, EMB // 4), lambda i: (0, 0)),
            pl.BlockSpec((EMB // 4, 1), lambda i: (0, 0)),
            pl.BlockSpec((EMB // 4, NUM_TASK), lambda i: (0, 0)),
            pl.BlockSpec((1, NUM_TASK), lambda i: (0, 0)),
        ],
        out_specs=pl.BlockSpec((NUM_GRAPHS, NUM_TASK), lambda i: (0, 0)),
        out_shape=jax.ShapeDtypeStruct((NUM_GRAPHS, NUM_TASK), jnp.float32),
        scratch_shapes=[pltpu.VMEM((EMB + 1, NUM_GRAPHS), jnp.float32)],
    )(batch_row, hT, W1, b1c, W2, b2c, W3, b3r)


def kernel(x, edge_index, batch, params):
    p = params
    src, dst = edge_index[0], edge_index[1]

    x_pad = jnp.zeros((N_PAD, 3), jnp.float32).at[:N_NODES].set(x)
    b_col = p["b_emb"][:, None]
    hT = _emb_call(x_pad, p["W_emb"], b_col)

    ones_e = jnp.ones((N_EDGES,), jnp.float32)
    deg = jax.ops.segment_sum(ones_e, dst, N_NODES)
    deg_row = jnp.zeros((1, N_PAD), jnp.float32).at[0, :N_NODES].set(deg)

    for l in range(NUM_LAYER):
        scale = p["bn_g"][l] / jnp.sqrt(p["bn_rv"][l] + 1e-5)
        shift = p["bn_b"][l] - p["bn_rm"][l] * scale
        WcT = p["conv_W"][l].T
        wT_stack = jnp.concatenate(
            [WcT[:, 0:4 * EMB], WcT[:, 4 * EMB:8 * EMB], WcT[:, 8 * EMB:12 * EMB]],
            axis=0) * jnp.tile(scale, 3)[:, None]
        bcol = (p["conv_b"][l] * scale + shift)[:, None]

        h = hT[:, :N_NODES].T
        msg = h[src]
        s = jax.ops.segment_sum(msg, dst, N_NODES)
        sq = jax.ops.segment_sum(msg * msg, dst, N_NODES)
        mn = jax.ops.segment_min(msg, dst, N_NODES)
        mx = jax.ops.segment_max(msg, dst, N_NODES)
        aggT = jnp.zeros((4 * EMB, N_PAD), jnp.float32)
        aggT = aggT.at[:, :N_NODES].set(
            jnp.concatenate([s.T, mn.T, mx.T, sq.T], axis=0))

        hT = _combine_call(deg_row, aggT, wT_stack, bcol, hT)

    batch_row = jnp.full((1, N_PAD), NUM_GRAPHS, jnp.int32).at[0, :N_NODES].set(batch)
    out = _pool_call(batch_row, hT, p["W1"], p["b1"][:, None],
                     p["W2"], p["b2"][:, None], p["W3"], p["b3"][None, :])
    return out


# speedup vs baseline: 1.6486x; 1.6486x over previous
import functools
import numpy as np
import jax
import jax.numpy as jnp
from jax import lax
from jax.experimental import pallas as pl
from jax.experimental.pallas import tpu as pltpu
from jax.experimental.pallas import tpu_sc as plsc

N_NODES = 10000
N_PAD = 10240
EMB = 128
NUM_LAYER = 4
NUM_TASK = 10
NUM_GRAPHS = 128
N_EDGES = 320000

_DEG_HIST = np.concatenate([np.zeros(32, np.float32), np.array([10000.0], np.float32)])
_B = np.arange(_DEG_HIST.shape[0], dtype=np.float32)
AVG_LOG = float((np.log(_B + 1.0) * _DEG_HIST).sum() / _DEG_HIST.sum())

NB = 1024
N_BLK = N_PAD // NB


def _emb_body(x_ref, w_ref, b_ref, out_ref):
    out_ref[...] = jax.lax.dot_general(
        w_ref[...], x_ref[...], (((0,), (1,)), ((), ())),
        preferred_element_type=jnp.float32) + b_ref[...]


def _emb_call(x_pad, W_emb, b_col):
    return pl.pallas_call(
        _emb_body,
        grid=(N_BLK,),
        in_specs=[
            pl.BlockSpec((NB, 3), lambda i: (i, 0)),
            pl.BlockSpec((3, EMB), lambda i: (0, 0)),
            pl.BlockSpec((EMB, 1), lambda i: (0, 0)),
        ],
        out_specs=pl.BlockSpec((EMB, NB), lambda i: (0, i)),
        out_shape=jax.ShapeDtypeStruct((EMB, N_PAD), jnp.float32),
    )(x_pad, W_emb, b_col)


def _combine_body(deg_ref, agg_ref, w_ref, b_ref, h_ref, out_ref):
    deg = deg_ref[...]
    degc = jnp.maximum(deg, 1.0)
    s = agg_ref[0:EMB, :]
    mn = agg_ref[EMB:2 * EMB, :]
    mx = agg_ref[2 * EMB:3 * EMB, :]
    sq = agg_ref[3 * EMB:4 * EMB, :]
    mean = s / degc
    msq = sq / degc
    std = jnp.sqrt(jnp.maximum(msq - mean * mean, 0.0) + 1e-5)
    has = deg > 0.0
    mn = jnp.where(has, mn, 0.0)
    mx = jnp.where(has, mx, 0.0)
    aggfix = jnp.concatenate([mean, mn, mx, std], axis=0)
    A = jax.lax.dot_general(
        w_ref[...], aggfix, (((1,), (0,)), ((), ())),
        preferred_element_type=jnp.float32)
    logd = jnp.log(deg + 1.0)
    s1 = logd / AVG_LOG
    s2 = jnp.where(logd > 0.0, AVG_LOG / jnp.maximum(logd, 1e-20), 0.0)
    c = A[0:EMB, :] + s1 * A[EMB:2 * EMB, :] + s2 * A[2 * EMB:3 * EMB, :] + b_ref[...]
    out_ref[...] = jnp.maximum(c, 0.0) + h_ref[...]


def _combine_call(deg_row, aggT, wT_stack, b_col, hT):
    return pl.pallas_call(
        _combine_body,
        grid=(N_BLK,),
        in_specs=[
            pl.BlockSpec((1, NB), lambda i: (0, i)),
            pl.BlockSpec((4 * EMB, NB), lambda i: (0, i)),
            pl.BlockSpec((3 * EMB, 4 * EMB), lambda i: (0, 0)),
            pl.BlockSpec((EMB, 1), lambda i: (0, 0)),
            pl.BlockSpec((EMB, NB), lambda i: (0, i)),
        ],
        out_specs=pl.BlockSpec((EMB, NB), lambda i: (0, i)),
        out_shape=jax.ShapeDtypeStruct((EMB, N_PAD), jnp.float32),
    )(deg_row, aggT, wT_stack, b_col, hT)


def _pool_body(batch_ref, h_ref, w1_ref, b1_ref, w2_ref, b2_ref, w3_ref,
               b3_ref, out_ref, acc):
    i = pl.program_id(0)

    @pl.when(i == 0)
    def _():
        acc[...] = jnp.zeros_like(acc)

    b = batch_ref[...]
    gids = jax.lax.broadcasted_iota(jnp.int32, (NUM_GRAPHS, NB), 0)
    M = (b == gids).astype(jnp.float32)
    h_ext = jnp.concatenate(
        [h_ref[...], jnp.ones((1, NB), jnp.float32)], axis=0)
    acc[...] += jax.lax.dot_general(
        h_ext, M, (((1,), (1,)), ((), ())),
        preferred_element_type=jnp.float32)

    @pl.when(i == pl.num_programs(0) - 1)
    def _():
        a = acc[...]
        hgm = a[0:EMB, :] / jnp.maximum(a[EMB:EMB + 1, :], 1.0)
        z1 = jnp.maximum(jax.lax.dot_general(
            w1_ref[...], hgm, (((0,), (0,)), ((), ())),
            preferred_element_type=jnp.float32) + b1_ref[...], 0.0)
        z2 = jnp.maximum(jax.lax.dot_general(
            w2_ref[...], z1, (((0,), (0,)), ((), ())),
            preferred_element_type=jnp.float32) + b2_ref[...], 0.0)
        out = jax.lax.dot_general(
            z2, w3_ref[...], (((0,), (0,)), ((), ())),
            preferred_element_type=jnp.float32) + b3_ref[...]
        out_ref[...] = out


def _pool_call(batch_row, hT, W1, b1c, W2, b2c, W3, b3r):
    return pl.pallas_call(
        _pool_body,
        grid=(N_BLK,),
        in_specs=[
            pl.BlockSpec((1, NB), lambda i: (0, i)),
            pl.BlockSpec((EMB, NB), lambda i: (0, i)),
            pl.BlockSpec((EMB, EMB // 2), lambda i: (0, 0)),
            pl.BlockSpec((EMB // 2, 1), lambda i: (0, 0)),
            pl.BlockSpec((EMB // 2, EMB // 4), lambda i: (0, 0)),
            pl.BlockSpec((EMB // 4, 1), lambda i: (0, 0)),
            pl.BlockSpec((EMB // 4, NUM_TASK), lambda i: (0, 0)),
            pl.BlockSpec((1, NUM_TASK), lambda i: (0, 0)),
        ],
        out_specs=pl.BlockSpec((NUM_GRAPHS, NUM_TASK), lambda i: (0, 0)),
        out_shape=jax.ShapeDtypeStruct((NUM_GRAPHS, NUM_TASK), jnp.float32),
        scratch_shapes=[pltpu.VMEM((EMB + 1, NUM_GRAPHS), jnp.float32)],
    )(batch_row, hT, W1, b1c, W2, b2c, W3, b3r)


NW = 32
FPP = 2
NPASS = EMB // (NW * FPP)
CHUNK = 2000
NCHUNK = N_EDGES // CHUNK
GROUPS = CHUNK // 16
BIG = 3.0e38


def _permute(v, idx):
    return lax.gather(
        v, idx[:, None],
        lax.GatherDimensionNumbers(offset_dims=(), collapsed_slice_dims=(0,),
                                   start_index_map=(0,)),
        (1,), mode=lax.GatherScatterMode.PROMISE_IN_BOUNDS)


def _sc_reduce_body(with_deg, hT_hbm, src_hbm, dst_hbm, agg_hbm, deg_hbm,
                    h_v, acc_s, acc_q, acc_mn, acc_mx, deg_v,
                    sbuf0, sbuf1, dbuf0, dbuf1, sem0, sem1):
    c = lax.axis_index("c")
    s = lax.axis_index("s")
    wid = s * 2 + c

    iota = lax.iota(jnp.int32, 16)
    ones16 = jnp.ones((16,), jnp.float32)
    shift_idx = [jnp.maximum(iota - k, 0) for k in (1, 2, 4, 8)]
    shift_ge = [iota >= k for k in (1, 2, 4, 8)]
    nxt_idx = jnp.minimum(iota + 1, 15)
    is15 = iota == 15
    sems = (sem0, sem1)
    sbufs = (sbuf0, sbuf1)
    dbufs = (dbuf0, dbuf1)

    for p in range(NPASS):
        f0 = p * (NW * FPP) + wid * FPP

        for j in range(FPP):
            pltpu.sync_copy(hT_hbm.at[f0 + j], h_v.at[j])

        def _zero(g, _):
            sl = pl.ds(g * 16, 16)
            for j in range(FPP):
                acc_s[j, sl] = jnp.zeros((16,), jnp.float32)
                acc_q[j, sl] = jnp.zeros((16,), jnp.float32)
                acc_mn[j, sl] = jnp.full((16,), BIG, jnp.float32)
                acc_mx[j, sl] = jnp.full((16,), -BIG, jnp.float32)
            if with_deg and p == 0:
                deg_v[sl] = jnp.zeros((16,), jnp.float32)
            return 0
        lax.fori_loop(0, N_PAD // 16, _zero, 0)

        for b in range(2):
            sl = pl.ds(b * CHUNK, CHUNK)
            pltpu.async_copy(src_hbm.at[sl], sbufs[b], sems[b])
            pltpu.async_copy(dst_hbm.at[sl], dbufs[b], sems[b])

        def _make_group(b):
          def _group(g, carry):
            sl = pl.ds(g * 16, 16)
            d16 = dbufs[b][sl]
            s16 = sbufs[b][sl]
            sd, ss = plsc.sort_key_val(d16, s16)
            if with_deg and p == 0:
                plsc.addupdate_scatter(deg_v, [sd], ones16)
            eqs = [(sd == _permute(sd, ix)) & ge
                   for ix, ge in zip(shift_idx, shift_ge)]
            m_end = (sd != _permute(sd, nxt_idx)) | is15
            for j in range(FPP):
                jf = jnp.full((16,), j, jnp.int32)
                val = plsc.load_gather(h_v, [jf, ss])
                plsc.addupdate_scatter(acc_s, [jf, sd], val)
                plsc.addupdate_scatter(acc_q, [jf, sd], val * val)
                mn = val
                mx = val
                for ix, eq in zip(shift_idx, eqs):
                    mn = jnp.where(eq, jnp.minimum(mn, _permute(mn, ix)), mn)
                    mx = jnp.where(eq, jnp.maximum(mx, _permute(mx, ix)), mx)
                cur = plsc.load_gather(acc_mn, [jf, sd], mask=m_end)
                plsc.store_scatter(acc_mn, [jf, sd], jnp.minimum(cur, mn),
                                   mask=m_end)
                cur = plsc.load_gather(acc_mx, [jf, sd], mask=m_end)
                plsc.store_scatter(acc_mx, [jf, sd], jnp.maximum(cur, mx),
                                   mask=m_end)
            return carry
          return _group

        groups = (_make_group(0), _make_group(1))

        def _chunk_pair(ci2, _):
            for b in range(2):
                ci = ci2 * 2 + b
                pltpu.make_async_copy(src_hbm.at[pl.ds(0, CHUNK)],
                                      sbufs[b], sems[b]).wait()
                pltpu.make_async_copy(dst_hbm.at[pl.ds(0, CHUNK)],
                                      dbufs[b], sems[b]).wait()
                lax.fori_loop(0, GROUPS, groups[b], 0)

                @pl.when(ci + 2 < NCHUNK)
                def _():
                    sl = pl.ds((ci + 2) * CHUNK, CHUNK)
                    pltpu.async_copy(src_hbm.at[sl], sbufs[b], sems[b])
                    pltpu.async_copy(dst_hbm.at[sl], dbufs[b], sems[b])
            return 0
        lax.fori_loop(0, NCHUNK // 2, _chunk_pair, 0)

        for j in range(FPP):
            f = f0 + j
            pltpu.sync_copy(acc_s.at[j], agg_hbm.at[0, f])
            pltpu.sync_copy(acc_mn.at[j], agg_hbm.at[1, f])
            pltpu.sync_copy(acc_mx.at[j], agg_hbm.at[2, f])
            pltpu.sync_copy(acc_q.at[j], agg_hbm.at[3, f])
        if with_deg and p == 0:
            @pl.when(wid == 0)
            def _():
                pltpu.sync_copy(deg_v, deg_hbm)


def _sc_reduce_nodeg_body(hT_hbm, src_hbm, dst_hbm, agg_hbm,
                          h_v, acc_s, acc_q, acc_mn, acc_mx,
                          sbuf0, sbuf1, dbuf0, dbuf1, sem0, sem1):
    _sc_reduce_body(False, hT_hbm, src_hbm, dst_hbm, agg_hbm, None,
                    h_v, acc_s, acc_q, acc_mn, acc_mx, None,
                    sbuf0, sbuf1, dbuf0, dbuf1, sem0, sem1)


def _sc_reduce_call(hT, src, dst, with_deg):
    mesh = plsc.VectorSubcoreMesh(core_axis_name="c", subcore_axis_name="s")
    out_type = [jax.ShapeDtypeStruct((4, EMB, N_PAD), jnp.float32)]
    scratch = [
        pltpu.VMEM((FPP, N_PAD), jnp.float32),
        pltpu.VMEM((FPP, N_PAD), jnp.float32),
        pltpu.VMEM((FPP, N_PAD), jnp.float32),
        pltpu.VMEM((FPP, N_PAD), jnp.float32),
        pltpu.VMEM((FPP, N_PAD), jnp.float32),
    ]
    if with_deg:
        out_type.append(jax.ShapeDtypeStruct((N_PAD,), jnp.float32))
        scratch.append(pltpu.VMEM((N_PAD,), jnp.float32))
        body = functools.partial(_sc_reduce_body, True)
    else:
        body = _sc_reduce_nodeg_body
    scratch += [
        pltpu.VMEM((CHUNK,), jnp.int32),
        pltpu.VMEM((CHUNK,), jnp.int32),
        pltpu.VMEM((CHUNK,), jnp.int32),
        pltpu.VMEM((CHUNK,), jnp.int32),
        pltpu.SemaphoreType.DMA,
        pltpu.SemaphoreType.DMA,
    ]
    fn = pl.kernel(
        body, out_type=out_type, mesh=mesh, scratch_types=scratch,
        compiler_params=pltpu.CompilerParams(needs_layout_passes=False))
    return fn(hT, src, dst)


def kernel(x, edge_index, batch, params):
    p = params
    src, dst = edge_index[0], edge_index[1]

    x_pad = jnp.zeros((N_PAD, 3), jnp.float32).at[:N_NODES].set(x)
    b_col = p["b_emb"][:, None]
    hT = _emb_call(x_pad, p["W_emb"], b_col)

    deg_row = None
    for l in range(NUM_LAYER):
        scale = p["bn_g"][l] / jnp.sqrt(p["bn_rv"][l] + 1e-5)
        shift = p["bn_b"][l] - p["bn_rm"][l] * scale
        WcT = p["conv_W"][l].T
        wT_stack = jnp.concatenate(
            [WcT[:, 0:4 * EMB], WcT[:, 4 * EMB:8 * EMB], WcT[:, 8 * EMB:12 * EMB]],
            axis=0) * jnp.tile(scale, 3)[:, None]
        bcol = (p["conv_b"][l] * scale + shift)[:, None]

        if l == 0:
            agg4, deg_pad = _sc_reduce_call(hT, src, dst, True)
            deg_row = deg_pad[None, :]
        else:
            (agg4,) = _sc_reduce_call(hT, src, dst, False)
        aggT = agg4.reshape(4 * EMB, N_PAD)

        hT = _combine_call(deg_row, aggT, wT_stack, bcol, hT)

    batch_row = jnp.full((1, N_PAD), NUM_GRAPHS, jnp.int32).at[0, :N_NODES].set(batch)
    out = _pool_call(batch_row, hT, p["W1"], p["b1"][:, None],
                     p["W2"], p["b2"][:, None], p["W3"], p["b3"][None, :])
    return out


# speedup vs baseline: 2.4779x; 1.5031x over previous
import functools
import numpy as np
import jax
import jax.numpy as jnp
from jax import lax
from jax.experimental import pallas as pl
from jax.experimental.pallas import tpu as pltpu
from jax.experimental.pallas import tpu_sc as plsc

N_NODES = 10000
N_PAD = 10240
EMB = 128
NUM_LAYER = 4
NUM_TASK = 10
NUM_GRAPHS = 128
N_EDGES = 320000

_DEG_HIST = np.concatenate([np.zeros(32, np.float32), np.array([10000.0], np.float32)])
_B = np.arange(_DEG_HIST.shape[0], dtype=np.float32)
AVG_LOG = float((np.log(_B + 1.0) * _DEG_HIST).sum() / _DEG_HIST.sum())

NB = 1024
N_BLK = N_PAD // NB


def _emb_body(x_ref, w_ref, b_ref, out_ref):
    out_ref[...] = jax.lax.dot_general(
        w_ref[...], x_ref[...], (((0,), (1,)), ((), ())),
        preferred_element_type=jnp.float32) + b_ref[...]


def _emb_call(x_pad, W_emb, b_col):
    return pl.pallas_call(
        _emb_body,
        grid=(N_BLK,),
        in_specs=[
            pl.BlockSpec((NB, 3), lambda i: (i, 0)),
            pl.BlockSpec((3, EMB), lambda i: (0, 0)),
            pl.BlockSpec((EMB, 1), lambda i: (0, 0)),
        ],
        out_specs=pl.BlockSpec((EMB, NB), lambda i: (0, i)),
        out_shape=jax.ShapeDtypeStruct((EMB, N_PAD), jnp.float32),
    )(x_pad, W_emb, b_col)


def _combine_body(deg_ref, agg_ref, w_ref, b_ref, h_ref, out_ref):
    deg = deg_ref[...]
    degc = jnp.maximum(deg, 1.0)
    s = agg_ref[0:EMB, :]
    mn = agg_ref[EMB:2 * EMB, :]
    mx = agg_ref[2 * EMB:3 * EMB, :]
    sq = agg_ref[3 * EMB:4 * EMB, :]
    mean = s / degc
    msq = sq / degc
    std = jnp.sqrt(jnp.maximum(msq - mean * mean, 0.0) + 1e-5)
    has = deg > 0.0
    mn = jnp.where(has, mn, 0.0)
    mx = jnp.where(has, mx, 0.0)
    aggfix = jnp.concatenate([mean, mn, mx, std], axis=0)
    A = jax.lax.dot_general(
        w_ref[...], aggfix, (((1,), (0,)), ((), ())),
        preferred_element_type=jnp.float32)
    logd = jnp.log(deg + 1.0)
    s1 = logd / AVG_LOG
    s2 = jnp.where(logd > 0.0, AVG_LOG / jnp.maximum(logd, 1e-20), 0.0)
    c = A[0:EMB, :] + s1 * A[EMB:2 * EMB, :] + s2 * A[2 * EMB:3 * EMB, :] + b_ref[...]
    out_ref[...] = jnp.maximum(c, 0.0) + h_ref[...]


def _combine_call(deg_row, aggT, wT_stack, b_col, hT):
    return pl.pallas_call(
        _combine_body,
        grid=(N_BLK,),
        in_specs=[
            pl.BlockSpec((1, NB), lambda i: (0, i)),
            pl.BlockSpec((4 * EMB, NB), lambda i: (0, i)),
            pl.BlockSpec((3 * EMB, 4 * EMB), lambda i: (0, 0)),
            pl.BlockSpec((EMB, 1), lambda i: (0, 0)),
            pl.BlockSpec((EMB, NB), lambda i: (0, i)),
        ],
        out_specs=pl.BlockSpec((EMB, NB), lambda i: (0, i)),
        out_shape=jax.ShapeDtypeStruct((EMB, N_PAD), jnp.float32),
    )(deg_row, aggT, wT_stack, b_col, hT)


def _pool_body(batch_ref, h_ref, w1_ref, b1_ref, w2_ref, b2_ref, w3_ref,
               b3_ref, out_ref, acc):
    i = pl.program_id(0)

    @pl.when(i == 0)
    def _():
        acc[...] = jnp.zeros_like(acc)

    b = batch_ref[...]
    gids = jax.lax.broadcasted_iota(jnp.int32, (NUM_GRAPHS, NB), 0)
    M = (b == gids).astype(jnp.float32)
    h_ext = jnp.concatenate(
        [h_ref[...], jnp.ones((1, NB), jnp.float32)], axis=0)
    acc[...] += jax.lax.dot_general(
        h_ext, M, (((1,), (1,)), ((), ())),
        preferred_element_type=jnp.float32)

    @pl.when(i == pl.num_programs(0) - 1)
    def _():
        a = acc[...]
        hgm = a[0:EMB, :] / jnp.maximum(a[EMB:EMB + 1, :], 1.0)
        z1 = jnp.maximum(jax.lax.dot_general(
            w1_ref[...], hgm, (((0,), (0,)), ((), ())),
            preferred_element_type=jnp.float32) + b1_ref[...], 0.0)
        z2 = jnp.maximum(jax.lax.dot_general(
            w2_ref[...], z1, (((0,), (0,)), ((), ())),
            preferred_element_type=jnp.float32) + b2_ref[...], 0.0)
        out = jax.lax.dot_general(
            z2, w3_ref[...], (((0,), (0,)), ((), ())),
            preferred_element_type=jnp.float32) + b3_ref[...]
        out_ref[...] = out


def _pool_call(batch_row, hT, W1, b1c, W2, b2c, W3, b3r):
    return pl.pallas_call(
        _pool_body,
        grid=(N_BLK,),
        in_specs=[
            pl.BlockSpec((1, NB), lambda i: (0, i)),
            pl.BlockSpec((EMB, NB), lambda i: (0, i)),
            pl.BlockSpec((EMB, EMB // 2), lambda i: (0, 0)),
            pl.BlockSpec((EMB // 2, 1), lambda i: (0, 0)),
            pl.BlockSpec((EMB // 2, EMB // 4), lambda i: (0, 0)),
            pl.BlockSpec((EMB // 4, 1), lambda i: (0, 0)),
            pl.BlockSpec((EMB // 4, NUM_TASK), lambda i: (0, 0)),
            pl.BlockSpec((1, NUM_TASK), lambda i: (0, 0)),
        ],
        out_specs=pl.BlockSpec((NUM_GRAPHS, NUM_TASK), lambda i: (0, 0)),
        out_shape=jax.ShapeDtypeStruct((NUM_GRAPHS, NUM_TASK), jnp.float32),
        scratch_shapes=[pltpu.VMEM((EMB + 1, NUM_GRAPHS), jnp.float32)],
    )(batch_row, hT, W1, b1c, W2, b2c, W3, b3r)


NW = 32
FPP = 2
NPASS = EMB // (NW * FPP)
CHUNK = 3200
NCHUNK = N_EDGES // CHUNK
GROUPS = CHUNK // 16
UNROLL = 2
HASH = 2048
BIG = 3.0e38


def _permute(v, idx):
    return lax.gather(
        v, idx[:, None],
        lax.GatherDimensionNumbers(offset_dims=(), collapsed_slice_dims=(0,),
                                   start_index_map=(0,)),
        (1,), mode=lax.GatherScatterMode.PROMISE_IN_BOUNDS)


def _sc_reduce_body(with_deg, hT_hbm, src_hbm, dst_hbm, agg_hbm, deg_hbm,
                    h_v, acc_s, acc_q, acc_mn, acc_mx, deg_v,
                    tmp_v, sbuf0, sbuf1, dbuf0, dbuf1, sem0, sem1):
    c = lax.axis_index("c")
    s = lax.axis_index("s")
    wid = s * 2 + c

    iota = lax.iota(jnp.int32, 16)
    ones16 = jnp.ones((16,), jnp.float32)
    shift_idx = [jnp.maximum(iota - k, 0) for k in (1, 2, 4, 8)]
    shift_ge = [iota >= k for k in (1, 2, 4, 8)]
    nxt_idx = jnp.minimum(iota + 1, 15)
    is15 = iota == 15
    sems = (sem0, sem1)
    sbufs = (sbuf0, sbuf1)
    dbufs = (dbuf0, dbuf1)

    for p in range(NPASS):
        f0 = p * (NW * FPP) + wid * FPP

        for j in range(FPP):
            pltpu.sync_copy(hT_hbm.at[f0 + j], h_v.at[j])

        def _zero(g, _):
            sl = pl.ds(g * 16, 16)
            for j in range(FPP):
                acc_s[j, sl] = jnp.zeros((16,), jnp.float32)
                acc_q[j, sl] = jnp.zeros((16,), jnp.float32)
                acc_mn[j, sl] = jnp.full((16,), BIG, jnp.float32)
                acc_mx[j, sl] = jnp.full((16,), -BIG, jnp.float32)
            if with_deg and p == 0:
                deg_v[sl] = jnp.zeros((16,), jnp.float32)
            return 0
        lax.fori_loop(0, N_PAD // 16, _zero, 0)

        for b in range(2):
            sl = pl.ds(b * CHUNK, CHUNK)
            pltpu.async_copy(src_hbm.at[sl], sbufs[b], sems[b])
            pltpu.async_copy(dst_hbm.at[sl], dbufs[b], sems[b])

        def _stage_a(b, g):
            sl = pl.ds(g * 16, 16)
            d16 = dbufs[b][sl]
            s16 = sbufs[b][sl]
            if with_deg and p == 0:
                plsc.addupdate_scatter(deg_v, [d16], ones16)
            ha = d16 & (HASH - 1)
            plsc.store_scatter(tmp_v, [ha], iota)
            rb = plsc.load_gather(tmp_v, [ha])
            ndup = plsc.all_reduce_population_count(rb != iota)[0]
            jfs, vals = [], []
            for j in range(FPP):
                jf = jnp.full((16,), j, jnp.int32)
                val = plsc.load_gather(h_v, [jf, s16])
                plsc.addupdate_scatter(acc_s, [jf, d16], val)
                plsc.addupdate_scatter(acc_q, [jf, d16], val * val)
                jfs.append(jf)
                vals.append(val)
            return d16, jfs, vals, ndup

        def _stage_b(state):
            d16, jfs, vals, ndup = state

            @pl.when(ndup == 0)
            def _fast():
                curs = []
                for j in range(FPP):
                    curs.append(plsc.load_gather(acc_mn, [jfs[j], d16]))
                    curs.append(plsc.load_gather(acc_mx, [jfs[j], d16]))
                for j in range(FPP):
                    plsc.store_scatter(acc_mn, [jfs[j], d16],
                                       jnp.minimum(curs[2 * j], vals[j]))
                    plsc.store_scatter(acc_mx, [jfs[j], d16],
                                       jnp.maximum(curs[2 * j + 1], vals[j]))

            @pl.when(ndup != 0)
            def _slow():
                sd, perm = plsc.sort_key_val(d16, iota)
                eqs = [(sd == _permute(sd, shift_idx[0])) & shift_ge[0]]
                eqs += [sd == _permute(sd, ix) for ix in shift_idx[1:]]
                m_end = (sd != _permute(sd, nxt_idx)) | is15
                for j in range(FPP):
                    mn = _permute(vals[j], perm)
                    mx = mn
                    for ix, eq in zip(shift_idx, eqs):
                        mn = jnp.where(eq, jnp.minimum(mn, _permute(mn, ix)), mn)
                        mx = jnp.where(eq, jnp.maximum(mx, _permute(mx, ix)), mx)
                    cur = plsc.load_gather(acc_mn, [jfs[j], sd], mask=m_end)
                    plsc.store_scatter(acc_mn, [jfs[j], sd],
                                       jnp.minimum(cur, mn), mask=m_end)
                    cur = plsc.load_gather(acc_mx, [jfs[j], sd], mask=m_end)
                    plsc.store_scatter(acc_mx, [jfs[j], sd],
                                       jnp.maximum(cur, mx), mask=m_end)

        def _make_group(b):
          def _group(gp, carry):
            states = [_stage_a(b, gp * UNROLL + u) for u in range(UNROLL)]
            for st in states:
                _stage_b(st)
            return carry
          return _group

        groups = (_make_group(0), _make_group(1))

        def _chunk_pair(ci2, _):
            for b in range(2):
                ci = ci2 * 2 + b
                pltpu.make_async_copy(src_hbm.at[pl.ds(0, CHUNK)],
                                      sbufs[b], sems[b]).wait()
                pltpu.make_async_copy(dst_hbm.at[pl.ds(0, CHUNK)],
                                      dbufs[b], sems[b]).wait()
                lax.fori_loop(0, GROUPS // UNROLL, groups[b], 0)

                @pl.when(ci + 2 < NCHUNK)
                def _():
                    sl = pl.ds((ci + 2) * CHUNK, CHUNK)
                    pltpu.async_copy(src_hbm.at[sl], sbufs[b], sems[b])
                    pltpu.async_copy(dst_hbm.at[sl], dbufs[b], sems[b])
            return 0
        lax.fori_loop(0, NCHUNK // 2, _chunk_pair, 0)

        for j in range(FPP):
            f = f0 + j
            pltpu.sync_copy(acc_s.at[j], agg_hbm.at[0, f])
            pltpu.sync_copy(acc_mn.at[j], agg_hbm.at[1, f])
            pltpu.sync_copy(acc_mx.at[j], agg_hbm.at[2, f])
            pltpu.sync_copy(acc_q.at[j], agg_hbm.at[3, f])
        if with_deg and p == 0:
            @pl.when(wid == 0)
            def _():
                pltpu.sync_copy(deg_v, deg_hbm)


def _sc_reduce_nodeg_body(hT_hbm, src_hbm, dst_hbm, agg_hbm,
                          h_v, acc_s, acc_q, acc_mn, acc_mx,
                          tmp_v, sbuf0, sbuf1, dbuf0, dbuf1, sem0, sem1):
    _sc_reduce_body(False, hT_hbm, src_hbm, dst_hbm, agg_hbm, None,
                    h_v, acc_s, acc_q, acc_mn, acc_mx, None,
                    tmp_v, sbuf0, sbuf1, dbuf0, dbuf1, sem0, sem1)


def _sc_reduce_call(hT, src, dst, with_deg):
    mesh = plsc.VectorSubcoreMesh(core_axis_name="c", subcore_axis_name="s")
    out_type = [jax.ShapeDtypeStruct((4, EMB, N_PAD), jnp.float32)]
    scratch = [
        pltpu.VMEM((FPP, N_PAD), jnp.float32),
        pltpu.VMEM((FPP, N_PAD), jnp.float32),
        pltpu.VMEM((FPP, N_PAD), jnp.float32),
        pltpu.VMEM((FPP, N_PAD), jnp.float32),
        pltpu.VMEM((FPP, N_PAD), jnp.float32),
    ]
    if with_deg:
        out_type.append(jax.ShapeDtypeStruct((N_PAD,), jnp.float32))
        scratch.append(pltpu.VMEM((N_PAD,), jnp.float32))
        body = functools.partial(_sc_reduce_body, True)
    else:
        body = _sc_reduce_nodeg_body
    scratch += [
        pltpu.VMEM((HASH,), jnp.int32),
        pltpu.VMEM((CHUNK,), jnp.int32),
        pltpu.VMEM((CHUNK,), jnp.int32),
        pltpu.VMEM((CHUNK,), jnp.int32),
        pltpu.VMEM((CHUNK,), jnp.int32),
        pltpu.SemaphoreType.DMA,
        pltpu.SemaphoreType.DMA,
    ]
    fn = pl.kernel(
        body, out_type=out_type, mesh=mesh, scratch_types=scratch,
        compiler_params=pltpu.CompilerParams(needs_layout_passes=False))
    return fn(hT, src, dst)


def kernel(x, edge_index, batch, params):
    p = params
    src, dst = edge_index[0], edge_index[1]

    x_pad = jnp.zeros((N_PAD, 3), jnp.float32).at[:N_NODES].set(x)
    b_col = p["b_emb"][:, None]
    hT = _emb_call(x_pad, p["W_emb"], b_col)

    deg_row = None
    for l in range(NUM_LAYER):
        scale = p["bn_g"][l] / jnp.sqrt(p["bn_rv"][l] + 1e-5)
        shift = p["bn_b"][l] - p["bn_rm"][l] * scale
        WcT = p["conv_W"][l].T
        wT_stack = jnp.concatenate(
            [WcT[:, 0:4 * EMB], WcT[:, 4 * EMB:8 * EMB], WcT[:, 8 * EMB:12 * EMB]],
            axis=0) * jnp.tile(scale, 3)[:, None]
        bcol = (p["conv_b"][l] * scale + shift)[:, None]

        if l == 0:
            agg4, deg_pad = _sc_reduce_call(hT, src, dst, True)
            deg_row = deg_pad[None, :]
        else:
            (agg4,) = _sc_reduce_call(hT, src, dst, False)
        aggT = agg4.reshape(4 * EMB, N_PAD)

        hT = _combine_call(deg_row, aggT, wT_stack, bcol, hT)

    batch_row = jnp.full((1, N_PAD), NUM_GRAPHS, jnp.int32).at[0, :N_NODES].set(batch)
    out = _pool_call(batch_row, hT, p["W1"], p["b1"][:, None],
                     p["W2"], p["b2"][:, None], p["W3"], p["b3"][None, :])
    return out


# speedup vs baseline: 2.5368x; 1.0238x over previous
import functools
import numpy as np
import jax
import jax.numpy as jnp
from jax import lax
from jax.experimental import pallas as pl
from jax.experimental.pallas import tpu as pltpu
from jax.experimental.pallas import tpu_sc as plsc

N_NODES = 10000
N_PAD = 10240
EMB = 128
NUM_LAYER = 4
NUM_TASK = 10
NUM_GRAPHS = 128
N_EDGES = 320000

_DEG_HIST = np.concatenate([np.zeros(32, np.float32), np.array([10000.0], np.float32)])
_B = np.arange(_DEG_HIST.shape[0], dtype=np.float32)
AVG_LOG = float((np.log(_B + 1.0) * _DEG_HIST).sum() / _DEG_HIST.sum())

NB = 1024
N_BLK = N_PAD // NB


def _emb_body(x_ref, w_ref, b_ref, out_ref):
    out_ref[...] = jax.lax.dot_general(
        w_ref[...], x_ref[...], (((0,), (1,)), ((), ())),
        preferred_element_type=jnp.float32) + b_ref[...]


def _emb_call(x_pad, W_emb, b_col):
    return pl.pallas_call(
        _emb_body,
        grid=(N_BLK,),
        in_specs=[
            pl.BlockSpec((NB, 3), lambda i: (i, 0)),
            pl.BlockSpec((3, EMB), lambda i: (0, 0)),
            pl.BlockSpec((EMB, 1), lambda i: (0, 0)),
        ],
        out_specs=pl.BlockSpec((EMB, NB), lambda i: (0, i)),
        out_shape=jax.ShapeDtypeStruct((EMB, N_PAD), jnp.float32),
    )(x_pad, W_emb, b_col)


def _combine_body(deg_ref, agg_ref, w_ref, b_ref, h_ref, out_ref):
    deg = deg_ref[...]
    degc = jnp.maximum(deg, 1.0)
    s = agg_ref[0:EMB, :]
    mn = agg_ref[EMB:2 * EMB, :]
    mx = agg_ref[2 * EMB:3 * EMB, :]
    sq = agg_ref[3 * EMB:4 * EMB, :]
    mean = s / degc
    msq = sq / degc
    std = jnp.sqrt(jnp.maximum(msq - mean * mean, 0.0) + 1e-5)
    has = deg > 0.0
    mn = jnp.where(has, mn, 0.0)
    mx = jnp.where(has, mx, 0.0)
    aggfix = jnp.concatenate([mean, mn, mx, std], axis=0)
    A = jax.lax.dot_general(
        w_ref[...], aggfix, (((1,), (0,)), ((), ())),
        preferred_element_type=jnp.float32)
    logd = jnp.log(deg + 1.0)
    s1 = logd / AVG_LOG
    s2 = jnp.where(logd > 0.0, AVG_LOG / jnp.maximum(logd, 1e-20), 0.0)
    c = A[0:EMB, :] + s1 * A[EMB:2 * EMB, :] + s2 * A[2 * EMB:3 * EMB, :] + b_ref[...]
    out_ref[...] = jnp.maximum(c, 0.0) + h_ref[...]


def _combine_call(deg_row, aggT, wT_stack, b_col, hT):
    return pl.pallas_call(
        _combine_body,
        grid=(N_BLK,),
        in_specs=[
            pl.BlockSpec((1, NB), lambda i: (0, i)),
            pl.BlockSpec((4 * EMB, NB), lambda i: (0, i)),
            pl.BlockSpec((3 * EMB, 4 * EMB), lambda i: (0, 0)),
            pl.BlockSpec((EMB, 1), lambda i: (0, 0)),
            pl.BlockSpec((EMB, NB), lambda i: (0, i)),
        ],
        out_specs=pl.BlockSpec((EMB, NB), lambda i: (0, i)),
        out_shape=jax.ShapeDtypeStruct((EMB, N_PAD), jnp.float32),
    )(deg_row, aggT, wT_stack, b_col, hT)


def _pool_body(batch_ref, h_ref, w1_ref, b1_ref, w2_ref, b2_ref, w3_ref,
               b3_ref, out_ref, acc):
    i = pl.program_id(0)

    @pl.when(i == 0)
    def _():
        acc[...] = jnp.zeros_like(acc)

    b = batch_ref[...]
    gids = jax.lax.broadcasted_iota(jnp.int32, (NUM_GRAPHS, NB), 0)
    M = (b == gids).astype(jnp.float32)
    h_ext = jnp.concatenate(
        [h_ref[...], jnp.ones((1, NB), jnp.float32)], axis=0)
    acc[...] += jax.lax.dot_general(
        h_ext, M, (((1,), (1,)), ((), ())),
        preferred_element_type=jnp.float32)

    @pl.when(i == pl.num_programs(0) - 1)
    def _():
        a = acc[...]
        hgm = a[0:EMB, :] / jnp.maximum(a[EMB:EMB + 1, :], 1.0)
        z1 = jnp.maximum(jax.lax.dot_general(
            w1_ref[...], hgm, (((0,), (0,)), ((), ())),
            preferred_element_type=jnp.float32) + b1_ref[...], 0.0)
        z2 = jnp.maximum(jax.lax.dot_general(
            w2_ref[...], z1, (((0,), (0,)), ((), ())),
            preferred_element_type=jnp.float32) + b2_ref[...], 0.0)
        out = jax.lax.dot_general(
            z2, w3_ref[...], (((0,), (0,)), ((), ())),
            preferred_element_type=jnp.float32) + b3_ref[...]
        out_ref[...] = out


def _pool_call(batch_row, hT, W1, b1c, W2, b2c, W3, b3r):
    return pl.pallas_call(
        _pool_body,
        grid=(N_BLK,),
        in_specs=[
            pl.BlockSpec((1, NB), lambda i: (0, i)),
            pl.BlockSpec((EMB, NB), lambda i: (0, i)),
            pl.BlockSpec((EMB, EMB // 2), lambda i: (0, 0)),
            pl.BlockSpec((EMB // 2, 1), lambda i: (0, 0)),
            pl.BlockSpec((EMB // 2, EMB // 4), lambda i: (0, 0)),
            pl.BlockSpec((EMB // 4, 1), lambda i: (0, 0)),
            pl.BlockSpec((EMB // 4, NUM_TASK), lambda i: (0, 0)),
            pl.BlockSpec((1, NUM_TASK), lambda i: (0, 0)),
        ],
        out_specs=pl.BlockSpec((NUM_GRAPHS, NUM_TASK), lambda i: (0, 0)),
        out_shape=jax.ShapeDtypeStruct((NUM_GRAPHS, NUM_TASK), jnp.float32),
        scratch_shapes=[pltpu.VMEM((EMB + 1, NUM_GRAPHS), jnp.float32)],
    )(batch_row, hT, W1, b1c, W2, b2c, W3, b3r)


NW = 32
FPP = 2
NPASS = EMB // (NW * FPP)
CHUNK = 3200
NCHUNK = N_EDGES // CHUNK
GROUPS = CHUNK // 16
UNROLL = 4
HASH = 4096
BIG = 3.0e38


def _permute(v, idx):
    return lax.gather(
        v, idx[:, None],
        lax.GatherDimensionNumbers(offset_dims=(), collapsed_slice_dims=(0,),
                                   start_index_map=(0,)),
        (1,), mode=lax.GatherScatterMode.PROMISE_IN_BOUNDS)


def _sc_reduce_body(with_deg, hT_hbm, src_hbm, dst_hbm, agg_hbm, deg_hbm,
                    h_v, acc_s, acc_q, acc_mn, acc_mx, deg_v,
                    tmp_v, sbuf0, sbuf1, dbuf0, dbuf1, sem0, sem1):
    c = lax.axis_index("c")
    s = lax.axis_index("s")
    wid = s * 2 + c

    iota = lax.iota(jnp.int32, 16)
    ones16 = jnp.ones((16,), jnp.float32)
    shift_idx = [jnp.maximum(iota - k, 0) for k in (1, 2, 4, 8)]
    shift_ge = [iota >= k for k in (1, 2, 4, 8)]
    nxt_idx = jnp.minimum(iota + 1, 15)
    is15 = iota == 15
    sems = (sem0, sem1)
    sbufs = (sbuf0, sbuf1)
    dbufs = (dbuf0, dbuf1)

    for p in range(NPASS):
        f0 = p * (NW * FPP) + wid * FPP

        for j in range(FPP):
            pltpu.sync_copy(hT_hbm.at[f0 + j], h_v.at[j])

        def _zero(g, _):
            sl = pl.ds(g * 16, 16)
            for j in range(FPP):
                acc_s[j, sl] = jnp.zeros((16,), jnp.float32)
                acc_q[j, sl] = jnp.zeros((16,), jnp.float32)
                acc_mn[j, sl] = jnp.full((16,), BIG, jnp.float32)
                acc_mx[j, sl] = jnp.full((16,), -BIG, jnp.float32)
            if with_deg and p == 0:
                deg_v[sl] = jnp.zeros((16,), jnp.float32)
            return 0
        lax.fori_loop(0, N_PAD // 16, _zero, 0)

        for b in range(2):
            sl = pl.ds(b * CHUNK, CHUNK)
            pltpu.async_copy(src_hbm.at[sl], sbufs[b], sems[b])
            pltpu.async_copy(dst_hbm.at[sl], dbufs[b], sems[b])

        def _stage_a(b, g):
            sl = pl.ds(g * 16, 16)
            d16 = dbufs[b][sl]
            s16 = sbufs[b][sl]
            if with_deg and p == 0:
                plsc.addupdate_scatter(deg_v, [d16], ones16)
            ha = d16 & (HASH - 1)
            plsc.store_scatter(tmp_v, [ha], iota)
            rb = plsc.load_gather(tmp_v, [ha])
            ndup = plsc.all_reduce_population_count(rb != iota)[0]
            jfs, vals = [], []
            for j in range(FPP):
                jf = jnp.full((16,), j, jnp.int32)
                val = plsc.load_gather(h_v, [jf, s16])
                plsc.addupdate_scatter(acc_s, [jf, d16], val)
                plsc.addupdate_scatter(acc_q, [jf, d16], val * val)
                jfs.append(jf)
                vals.append(val)
            return d16, jfs, vals, ndup

        def _stage_b(state):
            d16, jfs, vals, ndup = state

            @pl.when(ndup == 0)
            def _fast():
                curs = []
                for j in range(FPP):
                    curs.append(plsc.load_gather(acc_mn, [jfs[j], d16]))
                    curs.append(plsc.load_gather(acc_mx, [jfs[j], d16]))
                for j in range(FPP):
                    plsc.store_scatter(acc_mn, [jfs[j], d16],
                                       jnp.minimum(curs[2 * j], vals[j]))
                    plsc.store_scatter(acc_mx, [jfs[j], d16],
                                       jnp.maximum(curs[2 * j + 1], vals[j]))

            @pl.when(ndup != 0)
            def _slow():
                sd, perm = plsc.sort_key_val(d16, iota)
                eqs = [(sd == _permute(sd, shift_idx[0])) & shift_ge[0]]
                eqs += [sd == _permute(sd, ix) for ix in shift_idx[1:]]
                m_end = (sd != _permute(sd, nxt_idx)) | is15
                for j in range(FPP):
                    mn = _permute(vals[j], perm)
                    mx = mn
                    for ix, eq in zip(shift_idx, eqs):
                        mn = jnp.where(eq, jnp.minimum(mn, _permute(mn, ix)), mn)
                        mx = jnp.where(eq, jnp.maximum(mx, _permute(mx, ix)), mx)
                    cur = plsc.load_gather(acc_mn, [jfs[j], sd], mask=m_end)
                    plsc.store_scatter(acc_mn, [jfs[j], sd],
                                       jnp.minimum(cur, mn), mask=m_end)
                    cur = plsc.load_gather(acc_mx, [jfs[j], sd], mask=m_end)
                    plsc.store_scatter(acc_mx, [jfs[j], sd],
                                       jnp.maximum(cur, mx), mask=m_end)

        def _make_group(b):
          def _group(gp, carry):
            states = [_stage_a(b, gp * UNROLL + u) for u in range(UNROLL)]
            for st in states:
                _stage_b(st)
            return carry
          return _group

        groups = (_make_group(0), _make_group(1))

        def _chunk_pair(ci2, _):
            for b in range(2):
                ci = ci2 * 2 + b
                pltpu.make_async_copy(src_hbm.at[pl.ds(0, CHUNK)],
                                      sbufs[b], sems[b]).wait()
                pltpu.make_async_copy(dst_hbm.at[pl.ds(0, CHUNK)],
                                      dbufs[b], sems[b]).wait()
                lax.fori_loop(0, GROUPS // UNROLL, groups[b], 0)

                @pl.when(ci + 2 < NCHUNK)
                def _():
                    sl = pl.ds((ci + 2) * CHUNK, CHUNK)
                    pltpu.async_copy(src_hbm.at[sl], sbufs[b], sems[b])
                    pltpu.async_copy(dst_hbm.at[sl], dbufs[b], sems[b])
            return 0
        lax.fori_loop(0, NCHUNK // 2, _chunk_pair, 0)

        for j in range(FPP):
            f = f0 + j
            pltpu.sync_copy(acc_s.at[j], agg_hbm.at[0, f])
            pltpu.sync_copy(acc_mn.at[j], agg_hbm.at[1, f])
            pltpu.sync_copy(acc_mx.at[j], agg_hbm.at[2, f])
            pltpu.sync_copy(acc_q.at[j], agg_hbm.at[3, f])
        if with_deg and p == 0:
            @pl.when(wid == 0)
            def _():
                pltpu.sync_copy(deg_v, deg_hbm)


def _sc_reduce_nodeg_body(hT_hbm, src_hbm, dst_hbm, agg_hbm,
                          h_v, acc_s, acc_q, acc_mn, acc_mx,
                          tmp_v, sbuf0, sbuf1, dbuf0, dbuf1, sem0, sem1):
    _sc_reduce_body(False, hT_hbm, src_hbm, dst_hbm, agg_hbm, None,
                    h_v, acc_s, acc_q, acc_mn, acc_mx, None,
                    tmp_v, sbuf0, sbuf1, dbuf0, dbuf1, sem0, sem1)


def _sc_reduce_call(hT, src, dst, with_deg):
    mesh = plsc.VectorSubcoreMesh(core_axis_name="c", subcore_axis_name="s")
    out_type = [jax.ShapeDtypeStruct((4, EMB, N_PAD), jnp.float32)]
    scratch = [
        pltpu.VMEM((FPP, N_PAD), jnp.float32),
        pltpu.VMEM((FPP, N_PAD), jnp.float32),
        pltpu.VMEM((FPP, N_PAD), jnp.float32),
        pltpu.VMEM((FPP, N_PAD), jnp.float32),
        pltpu.VMEM((FPP, N_PAD), jnp.float32),
    ]
    if with_deg:
        out_type.append(jax.ShapeDtypeStruct((N_PAD,), jnp.float32))
        scratch.append(pltpu.VMEM((N_PAD,), jnp.float32))
        body = functools.partial(_sc_reduce_body, True)
    else:
        body = _sc_reduce_nodeg_body
    scratch += [
        pltpu.VMEM((HASH,), jnp.int32),
        pltpu.VMEM((CHUNK,), jnp.int32),
        pltpu.VMEM((CHUNK,), jnp.int32),
        pltpu.VMEM((CHUNK,), jnp.int32),
        pltpu.VMEM((CHUNK,), jnp.int32),
        pltpu.SemaphoreType.DMA,
        pltpu.SemaphoreType.DMA,
    ]
    fn = pl.kernel(
        body, out_type=out_type, mesh=mesh, scratch_types=scratch,
        compiler_params=pltpu.CompilerParams(needs_layout_passes=False))
    return fn(hT, src, dst)


def kernel(x, edge_index, batch, params):
    p = params
    src, dst = edge_index[0], edge_index[1]

    x_pad = jnp.zeros((N_PAD, 3), jnp.float32).at[:N_NODES].set(x)
    b_col = p["b_emb"][:, None]
    hT = _emb_call(x_pad, p["W_emb"], b_col)

    deg_row = None
    for l in range(NUM_LAYER):
        scale = p["bn_g"][l] / jnp.sqrt(p["bn_rv"][l] + 1e-5)
        shift = p["bn_b"][l] - p["bn_rm"][l] * scale
        WcT = p["conv_W"][l].T
        wT_stack = jnp.concatenate(
            [WcT[:, 0:4 * EMB], WcT[:, 4 * EMB:8 * EMB], WcT[:, 8 * EMB:12 * EMB]],
            axis=0) * jnp.tile(scale, 3)[:, None]
        bcol = (p["conv_b"][l] * scale + shift)[:, None]

        if l == 0:
            agg4, deg_pad = _sc_reduce_call(hT, src, dst, True)
            deg_row = deg_pad[None, :]
        else:
            (agg4,) = _sc_reduce_call(hT, src, dst, False)
        aggT = agg4.reshape(4 * EMB, N_PAD)

        hT = _combine_call(deg_row, aggT, wT_stack, bcol, hT)

    batch_row = jnp.full((1, N_PAD), NUM_GRAPHS, jnp.int32).at[0, :N_NODES].set(batch)
    out = _pool_call(batch_row, hT, p["W1"], p["b1"][:, None],
                     p["W2"], p["b2"][:, None], p["W3"], p["b3"][None, :])
    return out


# speedup vs baseline: 3.2902x; 1.2970x over previous
import functools
import numpy as np
import jax
import jax.numpy as jnp
from jax import lax
from jax.experimental import pallas as pl
from jax.experimental.pallas import tpu as pltpu
from jax.experimental.pallas import tpu_sc as plsc

N_NODES = 10000
N_PAD = 10240
EMB = 128
NUM_LAYER = 4
NUM_TASK = 10
NUM_GRAPHS = 128
N_EDGES = 320000

_DEG_HIST = np.concatenate([np.zeros(32, np.float32), np.array([10000.0], np.float32)])
_B = np.arange(_DEG_HIST.shape[0], dtype=np.float32)
AVG_LOG = float((np.log(_B + 1.0) * _DEG_HIST).sum() / _DEG_HIST.sum())

NB = 1024
N_BLK = N_PAD // NB


def _t(x):
    return jnp.transpose(x)


def _emb_body(x_ref, w_ref, b_ref, out_ref, out2_ref):
    hT = jax.lax.dot_general(
        w_ref[...], x_ref[...], (((0,), (1,)), ((), ())),
        preferred_element_type=jnp.float32) + b_ref[...]
    out_ref[...] = hT
    hb = _t(hT)
    out2_ref[0, :, :] = hb
    out2_ref[1, :, :] = hb * hb


def _emb_call(x_pad, W_emb, b_col):
    return pl.pallas_call(
        _emb_body,
        grid=(N_BLK,),
        in_specs=[
            pl.BlockSpec((NB, 3), lambda i: (i, 0)),
            pl.BlockSpec((3, EMB), lambda i: (0, 0)),
            pl.BlockSpec((EMB, 1), lambda i: (0, 0)),
        ],
        out_specs=[
            pl.BlockSpec((EMB, NB), lambda i: (0, i)),
            pl.BlockSpec((2, NB, EMB), lambda i: (0, i, 0)),
        ],
        out_shape=[
            jax.ShapeDtypeStruct((EMB, N_PAD), jnp.float32),
            jax.ShapeDtypeStruct((2, N_PAD, EMB), jnp.float32),
        ],
    )(x_pad, W_emb, b_col)


def _combine_body(deg_ref, ss_ref, mm_ref, w_ref, b_ref, h_ref,
                  out_ref, out2_ref):
    deg = deg_ref[...]
    degc = jnp.maximum(deg, 1.0)
    sT = _t(ss_ref[0, :, :])
    sqT = _t(ss_ref[1, :, :])
    mean = sT / degc
    msq = sqT / degc
    std = jnp.sqrt(jnp.maximum(msq - mean * mean, 0.0) + 1e-5)
    has = deg > 0.0
    mn = jnp.where(has, mm_ref[0, :, :], 0.0)
    mx = jnp.where(has, mm_ref[1, :, :], 0.0)
    aggfix = jnp.concatenate([mean, mn, mx, std], axis=0)
    A = jax.lax.dot_general(
        w_ref[...], aggfix, (((1,), (0,)), ((), ())),
        preferred_element_type=jnp.float32)
    logd = jnp.log(deg + 1.0)
    s1 = logd / AVG_LOG
    s2 = jnp.where(logd > 0.0, AVG_LOG / jnp.maximum(logd, 1e-20), 0.0)
    c = A[0:EMB, :] + s1 * A[EMB:2 * EMB, :] + s2 * A[2 * EMB:3 * EMB, :] + b_ref[...]
    hT = jnp.maximum(c, 0.0) + h_ref[...]
    out_ref[...] = hT
    hb = _t(hT)
    out2_ref[0, :, :] = hb
    out2_ref[1, :, :] = hb * hb


def _combine_call(deg_row, ss_both, mnmx, wT_stack, b_col, hT):
    return pl.pallas_call(
        _combine_body,
        grid=(N_BLK,),
        in_specs=[
            pl.BlockSpec((1, NB), lambda i: (0, i)),
            pl.BlockSpec((2, NB, EMB), lambda i: (0, i, 0)),
            pl.BlockSpec((2, EMB, NB), lambda i: (0, 0, i)),
            pl.BlockSpec((3 * EMB, 4 * EMB), lambda i: (0, 0)),
            pl.BlockSpec((EMB, 1), lambda i: (0, 0)),
            pl.BlockSpec((EMB, NB), lambda i: (0, i)),
        ],
        out_specs=[
            pl.BlockSpec((EMB, NB), lambda i: (0, i)),
            pl.BlockSpec((2, NB, EMB), lambda i: (0, i, 0)),
        ],
        out_shape=[
            jax.ShapeDtypeStruct((EMB, N_PAD), jnp.float32),
            jax.ShapeDtypeStruct((2, N_PAD, EMB), jnp.float32),
        ],
    )(deg_row, ss_both, mnmx, wT_stack, b_col, hT)


def _pool_body(batch_ref, h_ref, w1_ref, b1_ref, w2_ref, b2_ref, w3_ref,
               b3_ref, out_ref, acc):
    i = pl.program_id(0)

    @pl.when(i == 0)
    def _():
        acc[...] = jnp.zeros_like(acc)

    b = batch_ref[...]
    gids = jax.lax.broadcasted_iota(jnp.int32, (NUM_GRAPHS, NB), 0)
    M = (b == gids).astype(jnp.float32)
    h_ext = jnp.concatenate(
        [h_ref[...], jnp.ones((1, NB), jnp.float32)], axis=0)
    acc[...] += jax.lax.dot_general(
        h_ext, M, (((1,), (1,)), ((), ())),
        preferred_element_type=jnp.float32)

    @pl.when(i == pl.num_programs(0) - 1)
    def _():
        a = acc[...]
        hgm = a[0:EMB, :] / jnp.maximum(a[EMB:EMB + 1, :], 1.0)
        z1 = jnp.maximum(jax.lax.dot_general(
            w1_ref[...], hgm, (((0,), (0,)), ((), ())),
            preferred_element_type=jnp.float32) + b1_ref[...], 0.0)
        z2 = jnp.maximum(jax.lax.dot_general(
            w2_ref[...], z1, (((0,), (0,)), ((), ())),
            preferred_element_type=jnp.float32) + b2_ref[...], 0.0)
        out = jax.lax.dot_general(
            z2, w3_ref[...], (((0,), (0,)), ((), ())),
            preferred_element_type=jnp.float32) + b3_ref[...]
        out_ref[...] = out


def _pool_call(batch_row, hT, W1, b1c, W2, b2c, W3, b3r):
    return pl.pallas_call(
        _pool_body,
        grid=(N_BLK,),
        in_specs=[
            pl.BlockSpec((1, NB), lambda i: (0, i)),
            pl.BlockSpec((EMB, NB), lambda i: (0, i)),
            pl.BlockSpec((EMB, EMB // 2), lambda i: (0, 0)),
            pl.BlockSpec((EMB // 2, 1), lambda i: (0, 0)),
            pl.BlockSpec((EMB // 2, EMB // 4), lambda i: (0, 0)),
            pl.BlockSpec((EMB // 4, 1), lambda i: (0, 0)),
            pl.BlockSpec((EMB // 4, NUM_TASK), lambda i: (0, 0)),
            pl.BlockSpec((1, NUM_TASK), lambda i: (0, 0)),
        ],
        out_specs=pl.BlockSpec((NUM_GRAPHS, NUM_TASK), lambda i: (0, 0)),
        out_shape=jax.ShapeDtypeStruct((NUM_GRAPHS, NUM_TASK), jnp.float32),
        scratch_shapes=[pltpu.VMEM((EMB + 1, NUM_GRAPHS), jnp.float32)],
    )(batch_row, hT, W1, b1c, W2, b2c, W3, b3r)


EPW_D = N_EDGES // 32


def _sc_deg_body(dst_hbm, out_hbm, acc, dbuf, sem0):
    c = lax.axis_index("c")
    s = lax.axis_index("s")
    wid = s * 2 + c
    ones16 = jnp.ones((16,), jnp.float32)

    def _zero(g, _):
        acc[pl.ds(g * 16, 16)] = jnp.zeros((16,), jnp.float32)
        return 0
    lax.fori_loop(0, N_PAD // 16, _zero, 0)

    pltpu.sync_copy(dst_hbm.at[pl.ds(wid * EPW_D, EPW_D)], dbuf)

    def _grp(g, _):
        d16 = dbuf[pl.ds(g * 16, 16)]
        plsc.addupdate_scatter(acc, [d16], ones16)
        return 0
    lax.fori_loop(0, EPW_D // 16, _grp, 0)
    pltpu.sync_copy(acc, out_hbm.at[wid])


def _sc_deg_call(dst):
    mesh = plsc.VectorSubcoreMesh(core_axis_name="c", subcore_axis_name="s")
    fn = pl.kernel(
        _sc_deg_body,
        out_type=[jax.ShapeDtypeStruct((32, N_PAD), jnp.float32)],
        mesh=mesh,
        scratch_types=[
            pltpu.VMEM((N_PAD,), jnp.float32),
            pltpu.VMEM((EPW_D,), jnp.int32),
            pltpu.SemaphoreType.DMA,
        ],
        compiler_params=pltpu.CompilerParams(needs_layout_passes=False))
    (out,) = fn(dst)
    return out


def _degsum_body(dp_ref, out_ref):
    out_ref[...] = jnp.sum(dp_ref[...], axis=0, keepdims=True)


def _degsum_call(deg_parts):
    return pl.pallas_call(
        _degsum_body,
        grid=(N_BLK,),
        in_specs=[pl.BlockSpec((32, NB), lambda i: (0, i))],
        out_specs=pl.BlockSpec((1, NB), lambda i: (0, i)),
        out_shape=jax.ShapeDtypeStruct((1, N_PAD), jnp.float32),
    )(deg_parts)


NSUB = 16
E2 = 80
EPW = N_EDGES // NSUB
NCH2 = EPW // E2
N_SP = 10112
ROWS_PW = N_SP // NSUB


def _sc_stream_body(hb_hbm, src_hbm, dst_hbm, zeros_hbm, out_hbm,
                    sidx0, sidx1, didx0, didx1, rows0, rows1, acc_sh,
                    gs0, gs1):
    c = lax.axis_index("c")
    s = lax.axis_index("s")
    sidxs = (sidx0, sidx1)
    didxs = (didx0, didx1)
    rows = (rows0, rows1)
    gsems = (gs0, gs1)
    table = hb_hbm.at[c]

    r0 = s * ROWS_PW
    pltpu.sync_copy(zeros_hbm, acc_sh.at[pl.ds(r0, ROWS_PW)])
    plsc.subcore_barrier()

    base = s * EPW

    for b in range(2):
        sl = pl.ds(base + b * E2, E2)
        pltpu.sync_copy(src_hbm.at[sl], sidxs[b])
        pltpu.sync_copy(dst_hbm.at[sl], didxs[b])
        pltpu.async_copy(table.at[sidxs[b]], rows[b], gsems[b])

    def _pair(ci2, _):
        for b in range(2):
            ci = ci2 * 2 + b
            pltpu.make_async_copy(table.at[sidxs[b]], rows[b],
                                  gsems[b]).wait()
            pltpu.sync_copy(rows[b], acc_sh.at[didxs[b]], add=True)

            @pl.when(ci + 2 < NCH2)
            def _():
                sl = pl.ds(base + (ci + 2) * E2, E2)
                pltpu.sync_copy(src_hbm.at[sl], sidxs[b])
                pltpu.sync_copy(dst_hbm.at[sl], didxs[b])
                pltpu.async_copy(table.at[sidxs[b]], rows[b], gsems[b])
        return 0
    lax.fori_loop(0, NCH2 // 2, _pair, 0)

    plsc.subcore_barrier()
    pltpu.sync_copy(acc_sh.at[pl.ds(r0, ROWS_PW)],
                    out_hbm.at[c].at[pl.ds(r0, ROWS_PW)])

    @pl.when(s == NSUB - 1)
    def _():
        pltpu.sync_copy(zeros_hbm.at[pl.ds(0, N_PAD - N_NODES)],
                        out_hbm.at[c].at[pl.ds(N_NODES, N_PAD - N_NODES)])


def _sc_stream_call(h_both, src, dst, zeros_hbm):
    mesh = plsc.VectorSubcoreMesh(core_axis_name="c", subcore_axis_name="s")
    fn = pl.kernel(
        _sc_stream_body,
        out_type=[jax.ShapeDtypeStruct((2, N_PAD, EMB), jnp.float32)],
        mesh=mesh,
        scratch_types=[
            pltpu.VMEM((E2,), jnp.int32),
            pltpu.VMEM((E2,), jnp.int32),
            pltpu.VMEM((E2,), jnp.int32),
            pltpu.VMEM((E2,), jnp.int32),
            pltpu.VMEM((E2, EMB), jnp.float32),
            pltpu.VMEM((E2, EMB), jnp.float32),
            pltpu.VMEM_SHARED((N_SP, EMB), jnp.float32),
            pltpu.SemaphoreType.DMA,
            pltpu.SemaphoreType.DMA,
        ],
        compiler_params=pltpu.CompilerParams(needs_layout_passes=False))
    (out,) = fn(h_both, src, dst, zeros_hbm)
    return out


NW = 32
FPP = 4
CHUNK = 1600
NCHUNK = N_EDGES // CHUNK
GROUPS = CHUNK // 16
UNROLL = 4
HASH = 1024
BIG = 3.0e38


def _permute(v, idx):
    return lax.gather(
        v, idx[:, None],
        lax.GatherDimensionNumbers(offset_dims=(), collapsed_slice_dims=(0,),
                                   start_index_map=(0,)),
        (1,), mode=lax.GatherScatterMode.PROMISE_IN_BOUNDS)


def _sc_lane_body(hT_hbm, src_hbm, dst_hbm, mm_hbm,
                  h_v, acc_mn, acc_mx, tmp_v,
                  sbuf0, sbuf1, dbuf0, dbuf1, sem0, sem1):
    c = lax.axis_index("c")
    s = lax.axis_index("s")
    wid = s * 2 + c

    iota = lax.iota(jnp.int32, 16)
    shift_idx = [jnp.maximum(iota - k, 0) for k in (1, 2, 4, 8)]
    ge1 = iota >= 1
    nxt_idx = jnp.minimum(iota + 1, 15)
    is15 = iota == 15
    sems = (sem0, sem1)
    sbufs = (sbuf0, sbuf1)
    dbufs = (dbuf0, dbuf1)

    f0 = wid * FPP
    for j in range(FPP):
        pltpu.sync_copy(hT_hbm.at[f0 + j], h_v.at[j])

    def _zero(g, _):
        sl = pl.ds(g * 16, 16)
        for j in range(FPP):
            acc_mn[j, sl] = jnp.full((16,), BIG, jnp.float32)
            acc_mx[j, sl] = jnp.full((16,), -BIG, jnp.float32)
        return 0
    lax.fori_loop(0, N_PAD // 16, _zero, 0)

    for b in range(2):
        sl = pl.ds(b * CHUNK, CHUNK)
        pltpu.async_copy(src_hbm.at[sl], sbufs[b], sems[b])
        pltpu.async_copy(dst_hbm.at[sl], dbufs[b], sems[b])

    def _stage_a(b, g):
        sl = pl.ds(g * 16, 16)
        d16 = dbufs[b][sl]
        s16 = sbufs[b][sl]
        ha = d16 & (HASH - 1)
        plsc.store_scatter(tmp_v, [ha], iota)
        rb = plsc.load_gather(tmp_v, [ha])
        ndup = plsc.all_reduce_population_count(rb != iota)[0]
        jfs, vals = [], []
        for j in range(FPP):
            jf = jnp.full((16,), j, jnp.int32)
            vals.append(plsc.load_gather(h_v, [jf, s16]))
            jfs.append(jf)
        return d16, jfs, vals, ndup

    def _stage_b(state):
        d16, jfs, vals, ndup = state

        @pl.when(ndup == 0)
        def _fast():
            curs = []
            for j in range(FPP):
                curs.append(plsc.load_gather(acc_mn, [jfs[j], d16]))
                curs.append(plsc.load_gather(acc_mx, [jfs[j], d16]))
            for j in range(FPP):
                plsc.store_scatter(acc_mn, [jfs[j], d16],
                                   jnp.minimum(curs[2 * j], vals[j]))
                plsc.store_scatter(acc_mx, [jfs[j], d16],
                                   jnp.maximum(curs[2 * j + 1], vals[j]))

        @pl.when(ndup != 0)
        def _slow():
            sd, perm = plsc.sort_key_val(d16, iota)
            eqs = [(sd == _permute(sd, shift_idx[0])) & ge1]
            eqs += [sd == _permute(sd, ix) for ix in shift_idx[1:]]
            m_end = (sd != _permute(sd, nxt_idx)) | is15
            for j in range(FPP):
                mn = _permute(vals[j], perm)
                mx = mn
                for ix, eq in zip(shift_idx, eqs):
                    mn = jnp.where(eq, jnp.minimum(mn, _permute(mn, ix)), mn)
                    mx = jnp.where(eq, jnp.maximum(mx, _permute(mx, ix)), mx)
                cur = plsc.load_gather(acc_mn, [jfs[j], sd], mask=m_end)
                plsc.store_scatter(acc_mn, [jfs[j], sd],
                                   jnp.minimum(cur, mn), mask=m_end)
                cur = plsc.load_gather(acc_mx, [jfs[j], sd], mask=m_end)
                plsc.store_scatter(acc_mx, [jfs[j], sd],
                                   jnp.maximum(cur, mx), mask=m_end)

    def _make_group(b):
        def _group(gp, carry):
            states = [_stage_a(b, gp * UNROLL + u) for u in range(UNROLL)]
            for st in states:
                _stage_b(st)
            return carry
        return _group

    groups = (_make_group(0), _make_group(1))

    def _chunk_pair(ci2, _):
        for b in range(2):
            ci = ci2 * 2 + b
            pltpu.make_async_copy(src_hbm.at[pl.ds(0, CHUNK)],
                                  sbufs[b], sems[b]).wait()
            pltpu.make_async_copy(dst_hbm.at[pl.ds(0, CHUNK)],
                                  dbufs[b], sems[b]).wait()
            lax.fori_loop(0, GROUPS // UNROLL, groups[b], 0)

            @pl.when(ci + 2 < NCHUNK)
            def _():
                sl = pl.ds((ci + 2) * CHUNK, CHUNK)
                pltpu.async_copy(src_hbm.at[sl], sbufs[b], sems[b])
                pltpu.async_copy(dst_hbm.at[sl], dbufs[b], sems[b])
        return 0
    lax.fori_loop(0, NCHUNK // 2, _chunk_pair, 0)

    for j in range(FPP):
        f = f0 + j
        pltpu.sync_copy(acc_mn.at[j], mm_hbm.at[0, f])
        pltpu.sync_copy(acc_mx.at[j], mm_hbm.at[1, f])


def _sc_lane_call(hT, src, dst):
    mesh = plsc.VectorSubcoreMesh(core_axis_name="c", subcore_axis_name="s")
    fn = pl.kernel(
        _sc_lane_body,
        out_type=[jax.ShapeDtypeStruct((2, EMB, N_PAD), jnp.float32)],
        mesh=mesh,
        scratch_types=[
            pltpu.VMEM((FPP, N_PAD), jnp.float32),
            pltpu.VMEM((FPP, N_PAD), jnp.float32),
            pltpu.VMEM((FPP, N_PAD), jnp.float32),
            pltpu.VMEM((HASH,), jnp.int32),
            pltpu.VMEM((CHUNK,), jnp.int32),
            pltpu.VMEM((CHUNK,), jnp.int32),
            pltpu.VMEM((CHUNK,), jnp.int32),
            pltpu.VMEM((CHUNK,), jnp.int32),
            pltpu.SemaphoreType.DMA,
            pltpu.SemaphoreType.DMA,
        ],
        compiler_params=pltpu.CompilerParams(needs_layout_passes=False))
    (out,) = fn(hT, src, dst)
    return out


def kernel(x, edge_index, batch, params):
    p = params
    src, dst = edge_index[0], edge_index[1]

    x_pad = jnp.zeros((N_PAD, 3), jnp.float32).at[:N_NODES].set(x)
    b_col = p["b_emb"][:, None]
    hT, h_both = _emb_call(x_pad, p["W_emb"], b_col)
    zeros_hbm = jnp.zeros((ROWS_PW, EMB), jnp.float32)
    deg_row = _degsum_call(_sc_deg_call(dst))

    for l in range(NUM_LAYER):
        scale = p["bn_g"][l] / jnp.sqrt(p["bn_rv"][l] + 1e-5)
        shift = p["bn_b"][l] - p["bn_rm"][l] * scale
        WcT = p["conv_W"][l].T
        wT_stack = jnp.concatenate(
            [WcT[:, 0:4 * EMB], WcT[:, 4 * EMB:8 * EMB], WcT[:, 8 * EMB:12 * EMB]],
            axis=0) * jnp.tile(scale, 3)[:, None]
        bcol = (p["conv_b"][l] * scale + shift)[:, None]

        ss_both = _sc_stream_call(h_both, src, dst, zeros_hbm)
        mnmx = _sc_lane_call(hT, src, dst)

        hT, h_both = _combine_call(deg_row, ss_both, mnmx, wT_stack, bcol, hT)

    batch_row = jnp.full((1, N_PAD), NUM_GRAPHS, jnp.int32).at[0, :N_NODES].set(batch)
    out = _pool_call(batch_row, hT, p["W1"], p["b1"][:, None],
                     p["W2"], p["b2"][:, None], p["W3"], p["b3"][None, :])
    return out


# speedup vs baseline: 3.3883x; 1.0298x over previous
import functools
import numpy as np
import jax
import jax.numpy as jnp
from jax import lax
from jax.experimental import pallas as pl
from jax.experimental.pallas import tpu as pltpu
from jax.experimental.pallas import tpu_sc as plsc

N_NODES = 10000
N_PAD = 10240
EMB = 128
NUM_LAYER = 4
NUM_TASK = 10
NUM_GRAPHS = 128
N_EDGES = 320000

_DEG_HIST = np.concatenate([np.zeros(32, np.float32), np.array([10000.0], np.float32)])
_B = np.arange(_DEG_HIST.shape[0], dtype=np.float32)
AVG_LOG = float((np.log(_B + 1.0) * _DEG_HIST).sum() / _DEG_HIST.sum())

NB = 1024
N_BLK = N_PAD // NB


def _t(x):
    return jnp.transpose(x)


def _emb_body(x_ref, w_ref, b_ref, out_ref, out2_ref):
    hT = jax.lax.dot_general(
        w_ref[...], x_ref[...], (((0,), (1,)), ((), ())),
        preferred_element_type=jnp.float32) + b_ref[...]
    out_ref[...] = hT
    hb = _t(hT)
    out2_ref[0, :, :] = hb
    out2_ref[1, :, :] = hb * hb


def _emb_call(x_pad, W_emb, b_col):
    return pl.pallas_call(
        _emb_body,
        grid=(N_BLK,),
        in_specs=[
            pl.BlockSpec((NB, 3), lambda i: (i, 0)),
            pl.BlockSpec((3, EMB), lambda i: (0, 0)),
            pl.BlockSpec((EMB, 1), lambda i: (0, 0)),
        ],
        out_specs=[
            pl.BlockSpec((EMB, NB), lambda i: (0, i)),
            pl.BlockSpec((2, NB, EMB), lambda i: (0, i, 0)),
        ],
        out_shape=[
            jax.ShapeDtypeStruct((EMB, N_PAD), jnp.float32),
            jax.ShapeDtypeStruct((2, N_PAD, EMB), jnp.float32),
        ],
    )(x_pad, W_emb, b_col)


def _combine_body(deg_ref, ss_ref, mm_ref, w_ref, b_ref, h_ref,
                  out_ref, out2_ref):
    deg = deg_ref[...]
    degc = jnp.maximum(deg, 1.0)
    sT = _t(ss_ref[0, :, :])
    sqT = _t(ss_ref[1, :, :])
    mean = sT / degc
    msq = sqT / degc
    std = jnp.sqrt(jnp.maximum(msq - mean * mean, 0.0) + 1e-5)
    has = deg > 0.0
    mn = jnp.where(has, mm_ref[0, :, :], 0.0)
    mx = jnp.where(has, mm_ref[1, :, :], 0.0)
    aggfix = jnp.concatenate([mean, mn, mx, std], axis=0)
    A = jax.lax.dot_general(
        w_ref[...], aggfix, (((1,), (0,)), ((), ())),
        preferred_element_type=jnp.float32)
    logd = jnp.log(deg + 1.0)
    s1 = logd / AVG_LOG
    s2 = jnp.where(logd > 0.0, AVG_LOG / jnp.maximum(logd, 1e-20), 0.0)
    c = A[0:EMB, :] + s1 * A[EMB:2 * EMB, :] + s2 * A[2 * EMB:3 * EMB, :] + b_ref[...]
    hT = jnp.maximum(c, 0.0) + h_ref[...]
    out_ref[...] = hT
    hb = _t(hT)
    out2_ref[0, :, :] = hb
    out2_ref[1, :, :] = hb * hb


def _combine_call(deg_row, ss_both, mnmx, wT_stack, b_col, hT):
    return pl.pallas_call(
        _combine_body,
        grid=(N_BLK,),
        in_specs=[
            pl.BlockSpec((1, NB), lambda i: (0, i)),
            pl.BlockSpec((2, NB, EMB), lambda i: (0, i, 0)),
            pl.BlockSpec((2, EMB, NB), lambda i: (0, 0, i)),
            pl.BlockSpec((3 * EMB, 4 * EMB), lambda i: (0, 0)),
            pl.BlockSpec((EMB, 1), lambda i: (0, 0)),
            pl.BlockSpec((EMB, NB), lambda i: (0, i)),
        ],
        out_specs=[
            pl.BlockSpec((EMB, NB), lambda i: (0, i)),
            pl.BlockSpec((2, NB, EMB), lambda i: (0, i, 0)),
        ],
        out_shape=[
            jax.ShapeDtypeStruct((EMB, N_PAD), jnp.float32),
            jax.ShapeDtypeStruct((2, N_PAD, EMB), jnp.float32),
        ],
    )(deg_row, ss_both, mnmx, wT_stack, b_col, hT)


def _pool_body(batch_ref, h_ref, w1_ref, b1_ref, w2_ref, b2_ref, w3_ref,
               b3_ref, out_ref, acc):
    i = pl.program_id(0)

    @pl.when(i == 0)
    def _():
        acc[...] = jnp.zeros_like(acc)

    b = batch_ref[...]
    gids = jax.lax.broadcasted_iota(jnp.int32, (NUM_GRAPHS, NB), 0)
    M = (b == gids).astype(jnp.float32)
    h_ext = jnp.concatenate(
        [h_ref[...], jnp.ones((1, NB), jnp.float32)], axis=0)
    acc[...] += jax.lax.dot_general(
        h_ext, M, (((1,), (1,)), ((), ())),
        preferred_element_type=jnp.float32)

    @pl.when(i == pl.num_programs(0) - 1)
    def _():
        a = acc[...]
        hgm = a[0:EMB, :] / jnp.maximum(a[EMB:EMB + 1, :], 1.0)
        z1 = jnp.maximum(jax.lax.dot_general(
            w1_ref[...], hgm, (((0,), (0,)), ((), ())),
            preferred_element_type=jnp.float32) + b1_ref[...], 0.0)
        z2 = jnp.maximum(jax.lax.dot_general(
            w2_ref[...], z1, (((0,), (0,)), ((), ())),
            preferred_element_type=jnp.float32) + b2_ref[...], 0.0)
        out = jax.lax.dot_general(
            z2, w3_ref[...], (((0,), (0,)), ((), ())),
            preferred_element_type=jnp.float32) + b3_ref[...]
        out_ref[...] = out


def _pool_call(batch_row, hT, W1, b1c, W2, b2c, W3, b3r):
    return pl.pallas_call(
        _pool_body,
        grid=(N_BLK,),
        in_specs=[
            pl.BlockSpec((1, NB), lambda i: (0, i)),
            pl.BlockSpec((EMB, NB), lambda i: (0, i)),
            pl.BlockSpec((EMB, EMB // 2), lambda i: (0, 0)),
            pl.BlockSpec((EMB // 2, 1), lambda i: (0, 0)),
            pl.BlockSpec((EMB // 2, EMB // 4), lambda i: (0, 0)),
            pl.BlockSpec((EMB // 4, 1), lambda i: (0, 0)),
            pl.BlockSpec((EMB // 4, NUM_TASK), lambda i: (0, 0)),
            pl.BlockSpec((1, NUM_TASK), lambda i: (0, 0)),
        ],
        out_specs=pl.BlockSpec((NUM_GRAPHS, NUM_TASK), lambda i: (0, 0)),
        out_shape=jax.ShapeDtypeStruct((NUM_GRAPHS, NUM_TASK), jnp.float32),
        scratch_shapes=[pltpu.VMEM((EMB + 1, NUM_GRAPHS), jnp.float32)],
    )(batch_row, hT, W1, b1c, W2, b2c, W3, b3r)


EPW_D = N_EDGES // 32


def _sc_deg_body(dst_hbm, out_hbm, acc, dbuf, sem0):
    c = lax.axis_index("c")
    s = lax.axis_index("s")
    wid = s * 2 + c
    ones16 = jnp.ones((16,), jnp.float32)

    def _zero(g, _):
        acc[pl.ds(g * 16, 16)] = jnp.zeros((16,), jnp.float32)
        return 0
    lax.fori_loop(0, N_PAD // 16, _zero, 0)

    pltpu.sync_copy(dst_hbm.at[pl.ds(wid * EPW_D, EPW_D)], dbuf)

    def _grp(g, _):
        d16 = dbuf[pl.ds(g * 16, 16)]
        plsc.addupdate_scatter(acc, [d16], ones16)
        return 0
    lax.fori_loop(0, EPW_D // 16, _grp, 0)
    pltpu.sync_copy(acc, out_hbm.at[wid])


def _sc_deg_call(dst):
    mesh = plsc.VectorSubcoreMesh(core_axis_name="c", subcore_axis_name="s")
    fn = pl.kernel(
        _sc_deg_body,
        out_type=[jax.ShapeDtypeStruct((32, N_PAD), jnp.float32)],
        mesh=mesh,
        scratch_types=[
            pltpu.VMEM((N_PAD,), jnp.float32),
            pltpu.VMEM((EPW_D,), jnp.int32),
            pltpu.SemaphoreType.DMA,
        ],
        compiler_params=pltpu.CompilerParams(needs_layout_passes=False))
    (out,) = fn(dst)
    return out


def _degsum_body(dp_ref, out_ref):
    out_ref[...] = jnp.sum(dp_ref[...], axis=0, keepdims=True)


def _degsum_call(deg_parts):
    return pl.pallas_call(
        _degsum_body,
        grid=(N_BLK,),
        in_specs=[pl.BlockSpec((32, NB), lambda i: (0, i))],
        out_specs=pl.BlockSpec((1, NB), lambda i: (0, i)),
        out_shape=jax.ShapeDtypeStruct((1, N_PAD), jnp.float32),
    )(deg_parts)


NSUB = 16
E2 = 80
EPW = N_EDGES // NSUB
NCH2 = EPW // E2
N_SP = 10112
ROWS_PW = N_SP // NSUB


def _sc_stream_body(hb_hbm, src_hbm, dst_hbm, zeros_hbm, out_hbm,
                    sidx0, sidx1, didx0, didx1, rows0, rows1, acc_sh,
                    gs0, gs1):
    c = lax.axis_index("c")
    s = lax.axis_index("s")
    sidxs = (sidx0, sidx1)
    didxs = (didx0, didx1)
    rows = (rows0, rows1)
    gsems = (gs0, gs1)
    table = hb_hbm.at[c]

    r0 = s * ROWS_PW
    pltpu.sync_copy(zeros_hbm, acc_sh.at[pl.ds(r0, ROWS_PW)])
    plsc.subcore_barrier()

    base = s * EPW

    for b in range(2):
        sl = pl.ds(base + b * E2, E2)
        pltpu.sync_copy(src_hbm.at[sl], sidxs[b])
        pltpu.sync_copy(dst_hbm.at[sl], didxs[b])
        pltpu.async_copy(table.at[sidxs[b]], rows[b], gsems[b])

    def _pair(ci2, _):
        for b in range(2):
            ci = ci2 * 2 + b
            pltpu.make_async_copy(table.at[sidxs[b]], rows[b],
                                  gsems[b]).wait()
            pltpu.sync_copy(rows[b], acc_sh.at[didxs[b]], add=True)

            @pl.when(ci + 2 < NCH2)
            def _():
                sl = pl.ds(base + (ci + 2) * E2, E2)
                pltpu.sync_copy(src_hbm.at[sl], sidxs[b])
                pltpu.sync_copy(dst_hbm.at[sl], didxs[b])
                pltpu.async_copy(table.at[sidxs[b]], rows[b], gsems[b])
        return 0
    lax.fori_loop(0, NCH2 // 2, _pair, 0)

    plsc.subcore_barrier()
    pltpu.sync_copy(acc_sh.at[pl.ds(r0, ROWS_PW)],
                    out_hbm.at[c].at[pl.ds(r0, ROWS_PW)])

    @pl.when(s == NSUB - 1)
    def _():
        pltpu.sync_copy(zeros_hbm.at[pl.ds(0, N_PAD - N_NODES)],
                        out_hbm.at[c].at[pl.ds(N_NODES, N_PAD - N_NODES)])


def _sc_stream_call(h_both, src, dst, zeros_hbm):
    mesh = plsc.VectorSubcoreMesh(core_axis_name="c", subcore_axis_name="s")
    fn = pl.kernel(
        _sc_stream_body,
        out_type=[jax.ShapeDtypeStruct((2, N_PAD, EMB), jnp.float32)],
        mesh=mesh,
        scratch_types=[
            pltpu.VMEM((E2,), jnp.int32),
            pltpu.VMEM((E2,), jnp.int32),
            pltpu.VMEM((E2,), jnp.int32),
            pltpu.VMEM((E2,), jnp.int32),
            pltpu.VMEM((E2, EMB), jnp.float32),
            pltpu.VMEM((E2, EMB), jnp.float32),
            pltpu.VMEM_SHARED((N_SP, EMB), jnp.float32),
            pltpu.SemaphoreType.DMA,
            pltpu.SemaphoreType.DMA,
        ],
        compiler_params=pltpu.CompilerParams(needs_layout_passes=False))
    (out,) = fn(h_both, src, dst, zeros_hbm)
    return out


NW = 32
FPP = 4
CHUNK = 1280
NCHUNK = N_EDGES // CHUNK
GROUPS = CHUNK // 16
UNROLL = 4
HASH = 2048
BIG = 3.0e38


def _permute(v, idx):
    return lax.gather(
        v, idx[:, None],
        lax.GatherDimensionNumbers(offset_dims=(), collapsed_slice_dims=(0,),
                                   start_index_map=(0,)),
        (1,), mode=lax.GatherScatterMode.PROMISE_IN_BOUNDS)


def _sc_lane_body(hT_hbm, src_hbm, dst_hbm, mm_hbm,
                  h_v, acc_mn, acc_mx, tmp_v,
                  sbuf0, sbuf1, dbuf0, dbuf1, sem0, sem1):
    c = lax.axis_index("c")
    s = lax.axis_index("s")
    wid = s * 2 + c

    iota = lax.iota(jnp.int32, 16)
    shift_idx = [jnp.maximum(iota - k, 0) for k in (1, 2, 4, 8)]
    ge1 = iota >= 1
    nxt_idx = jnp.minimum(iota + 1, 15)
    is15 = iota == 15
    sems = (sem0, sem1)
    sbufs = (sbuf0, sbuf1)
    dbufs = (dbuf0, dbuf1)

    f0 = wid * FPP
    for j in range(FPP):
        pltpu.sync_copy(hT_hbm.at[f0 + j], h_v.at[j])

    def _zero(g, _):
        sl = pl.ds(g * 16, 16)
        for j in range(FPP):
            acc_mn[j, sl] = jnp.full((16,), BIG, jnp.float32)
            acc_mx[j, sl] = jnp.full((16,), -BIG, jnp.float32)
        return 0
    lax.fori_loop(0, N_PAD // 16, _zero, 0)

    for b in range(2):
        sl = pl.ds(b * CHUNK, CHUNK)
        pltpu.async_copy(src_hbm.at[sl], sbufs[b], sems[b])
        pltpu.async_copy(dst_hbm.at[sl], dbufs[b], sems[b])

    def _stage_a(b, g):
        sl = pl.ds(g * 16, 16)
        d16 = dbufs[b][sl]
        s16 = sbufs[b][sl]
        ha = d16 & (HASH - 1)
        plsc.store_scatter(tmp_v, [ha], iota)
        rb = plsc.load_gather(tmp_v, [ha])
        ndup = plsc.all_reduce_population_count(rb != iota)[0]
        jfs, vals = [], []
        for j in range(FPP):
            jf = jnp.full((16,), j, jnp.int32)
            vals.append(plsc.load_gather(h_v, [jf, s16]))
            jfs.append(jf)
        return d16, jfs, vals, ndup

    def _stage_b(state):
        d16, jfs, vals, ndup = state

        @pl.when(ndup == 0)
        def _fast():
            curs = []
            for j in range(FPP):
                curs.append(plsc.load_gather(acc_mn, [jfs[j], d16]))
                curs.append(plsc.load_gather(acc_mx, [jfs[j], d16]))
            for j in range(FPP):
                plsc.store_scatter(acc_mn, [jfs[j], d16],
                                   jnp.minimum(curs[2 * j], vals[j]))
                plsc.store_scatter(acc_mx, [jfs[j], d16],
                                   jnp.maximum(curs[2 * j + 1], vals[j]))

        @pl.when(ndup != 0)
        def _slow():
            sd, perm = plsc.sort_key_val(d16, iota)
            eqs = [(sd == _permute(sd, shift_idx[0])) & ge1]
            eqs += [sd == _permute(sd, ix) for ix in shift_idx[1:]]
            m_end = (sd != _permute(sd, nxt_idx)) | is15
            for j in range(FPP):
                mn = _permute(vals[j], perm)
                mx = mn
                for ix, eq in zip(shift_idx, eqs):
                    mn = jnp.where(eq, jnp.minimum(mn, _permute(mn, ix)), mn)
                    mx = jnp.where(eq, jnp.maximum(mx, _permute(mx, ix)), mx)
                cur = plsc.load_gather(acc_mn, [jfs[j], sd], mask=m_end)
                plsc.store_scatter(acc_mn, [jfs[j], sd],
                                   jnp.minimum(cur, mn), mask=m_end)
                cur = plsc.load_gather(acc_mx, [jfs[j], sd], mask=m_end)
                plsc.store_scatter(acc_mx, [jfs[j], sd],
                                   jnp.maximum(cur, mx), mask=m_end)

    def _make_group(b):
        def _group(gp, carry):
            states = [_stage_a(b, gp * UNROLL + u) for u in range(UNROLL)]
            for st in states:
                _stage_b(st)
            return carry
        return _group

    groups = (_make_group(0), _make_group(1))

    def _chunk_pair(ci2, _):
        for b in range(2):
            ci = ci2 * 2 + b
            pltpu.make_async_copy(src_hbm.at[pl.ds(0, CHUNK)],
                                  sbufs[b], sems[b]).wait()
            pltpu.make_async_copy(dst_hbm.at[pl.ds(0, CHUNK)],
                                  dbufs[b], sems[b]).wait()
            lax.fori_loop(0, GROUPS // UNROLL, groups[b], 0)

            @pl.when(ci + 2 < NCHUNK)
            def _():
                sl = pl.ds((ci + 2) * CHUNK, CHUNK)
                pltpu.async_copy(src_hbm.at[sl], sbufs[b], sems[b])
                pltpu.async_copy(dst_hbm.at[sl], dbufs[b], sems[b])
        return 0
    lax.fori_loop(0, NCHUNK // 2, _chunk_pair, 0)

    for j in range(FPP):
        f = f0 + j
        pltpu.sync_copy(acc_mn.at[j], mm_hbm.at[0, f])
        pltpu.sync_copy(acc_mx.at[j], mm_hbm.at[1, f])


def _sc_lane_call(hT, src, dst):
    mesh = plsc.VectorSubcoreMesh(core_axis_name="c", subcore_axis_name="s")
    fn = pl.kernel(
        _sc_lane_body,
        out_type=[jax.ShapeDtypeStruct((2, EMB, N_PAD), jnp.float32)],
        mesh=mesh,
        scratch_types=[
            pltpu.VMEM((FPP, N_PAD), jnp.float32),
            pltpu.VMEM((FPP, N_PAD), jnp.float32),
            pltpu.VMEM((FPP, N_PAD), jnp.float32),
            pltpu.VMEM((HASH,), jnp.int32),
            pltpu.VMEM((CHUNK,), jnp.int32),
            pltpu.VMEM((CHUNK,), jnp.int32),
            pltpu.VMEM((CHUNK,), jnp.int32),
            pltpu.VMEM((CHUNK,), jnp.int32),
            pltpu.SemaphoreType.DMA,
            pltpu.SemaphoreType.DMA,
        ],
        compiler_params=pltpu.CompilerParams(needs_layout_passes=False))
    (out,) = fn(hT, src, dst)
    return out


def kernel(x, edge_index, batch, params):
    p = params
    src, dst = edge_index[0], edge_index[1]

    x_pad = jnp.zeros((N_PAD, 3), jnp.float32).at[:N_NODES].set(x)
    b_col = p["b_emb"][:, None]
    hT, h_both = _emb_call(x_pad, p["W_emb"], b_col)
    zeros_hbm = jnp.zeros((ROWS_PW, EMB), jnp.float32)
    deg_row = _degsum_call(_sc_deg_call(dst))

    for l in range(NUM_LAYER):
        scale = p["bn_g"][l] / jnp.sqrt(p["bn_rv"][l] + 1e-5)
        shift = p["bn_b"][l] - p["bn_rm"][l] * scale
        WcT = p["conv_W"][l].T
        wT_stack = jnp.concatenate(
            [WcT[:, 0:4 * EMB], WcT[:, 4 * EMB:8 * EMB], WcT[:, 8 * EMB:12 * EMB]],
            axis=0) * jnp.tile(scale, 3)[:, None]
        bcol = (p["conv_b"][l] * scale + shift)[:, None]

        ss_both = _sc_stream_call(h_both, src, dst, zeros_hbm)
        mnmx = _sc_lane_call(hT, src, dst)

        hT, h_both = _combine_call(deg_row, ss_both, mnmx, wT_stack, bcol, hT)

    batch_row = jnp.full((1, N_PAD), NUM_GRAPHS, jnp.int32).at[0, :N_NODES].set(batch)
    out = _pool_call(batch_row, hT, p["W1"], p["b1"][:, None],
                     p["W2"], p["b2"][:, None], p["W3"], p["b3"][None, :])
    return out
